# baseline (speedup 1.0000x reference)
import functools

import jax
import jax.numpy as jnp
from jax import lax
from jax.experimental import pallas as pl
from jax.experimental.pallas import tpu as pltpu
from jax.experimental.pallas import tpu_sc as plsc

_Q = 4
_D = 128
_B = 16384
_NC = 2
_NS = 16
_NW = _NC * _NS
_RPW = _B // _NW
_VEC = 16
_CH = 128
_NG = _D // 4
_PITCH = _D + 1


def _sc_body(x_hbm, t4_hbm, out_hbm, xb0, xb1, tbuf, res, sem0, sem1):
    wid = lax.axis_index("s") * _NC + lax.axis_index("c")
    base = wid * _RPW
    pltpu.sync_copy(t4_hbm, tbuf)

    xbufs = (xb0, xb1)
    sems = (sem0, sem1)

    def chunk_copy(g):
        return pltpu.make_async_copy(
            x_hbm.at[pl.ds(base + g * _CH, _CH), :],
            xbufs[g % 2].at[:, 0:_D], sems[g % 2])

    chunk_copy(0).start()
    chunk_copy(1).start()

    lanes = lax.iota(jnp.int32, _VEC)

    def make_blk(buf, res_base):
        def blk_body(k, _):
            rowv = lanes + k * _VEC
            acc = jnp.zeros((_VEC,), jnp.float32)
            for g in range(_NG):
                xa = plsc.load_gather(buf, [rowv, lanes * 0 + 4 * g])
                xb = plsc.load_gather(buf, [rowv, lanes * 0 + (4 * g + 1)])
                xc = plsc.load_gather(buf, [rowv, lanes * 0 + (4 * g + 2)])
                xd = plsc.load_gather(buf, [rowv, lanes * 0 + (4 * g + 3)])
                idx = ((xa * 4 + xb) * 4 + xc) * 4 + (xd + g * 256)
                acc = acc + plsc.load_gather(tbuf, [idx])
            res[pl.ds(res_base + k * _VEC, _VEC)] = acc
            return _
        return blk_body

    nch = _RPW // _CH
    for g in range(nch):
        chunk_copy(g).wait()
        lax.fori_loop(0, _CH // _VEC, make_blk(xbufs[g % 2], g * _CH), 0)
        if g + 2 < nch:
            chunk_copy(g + 2).start()

    pltpu.sync_copy(res, out_hbm.at[pl.ds(base, _RPW)])


_sc_call = functools.partial(
    pl.kernel,
    out_type=jax.ShapeDtypeStruct((_B,), jnp.float32),
    mesh=plsc.VectorSubcoreMesh(core_axis_name="c", subcore_axis_name="s"),
    compiler_params=pltpu.CompilerParams(needs_layout_passes=False),
    scratch_types=[
        pltpu.VMEM((_CH, _PITCH), jnp.int32),
        pltpu.VMEM((_CH, _PITCH), jnp.int32),
        pltpu.VMEM((_NG * 256,), jnp.float32),
        pltpu.VMEM((_RPW,), jnp.float32),
        pltpu.SemaphoreType.DMA,
        pltpu.SemaphoreType.DMA,
    ],
)(_sc_body)


def _quad_table(category_parameters):
    lt = jnp.log(category_parameters * (1.0 - 0.2 * _Q) + 0.2)
    lr = lt.T.reshape(_NG, 4, _Q)
    c = jnp.arange(256)
    t4 = sum(lr[:, k, (c >> (6 - 2 * k)) & 3] for k in range(4))
    return t4.reshape(-1).astype(jnp.float32)


def kernel(x, category_parameters):
    out = _sc_call(x.astype(jnp.int32), _quad_table(category_parameters))
    return lax.stop_gradient(out[:, None])


# speedup vs baseline: 1.7517x; 1.7517x over previous
import functools

import jax
import jax.numpy as jnp
from jax import lax
from jax.experimental import pallas as pl
from jax.experimental.pallas import tpu as pltpu
from jax.experimental.pallas import tpu_sc as plsc

_Q = 4
_D = 128
_B = 16384
_NC = 2
_NS = 16
_NW = _NC * _NS
_RPW = _B // _NW
_VEC = 16
_CH = _RPW // 2


def _sc_body(x_hbm, lt_hbm, out_hbm, xbuf, tbuf, res, sem0, sem1):
    wid = lax.axis_index("s") * _NC + lax.axis_index("c")
    base = wid * _RPW
    pltpu.sync_copy(lt_hbm, tbuf)

    cp0 = pltpu.make_async_copy(
        x_hbm.at[pl.ds(base * _D, _CH * _D)],
        xbuf.at[pl.ds(0, _CH * _D)], sem0)
    cp1 = pltpu.make_async_copy(
        x_hbm.at[pl.ds((base + _CH) * _D, _CH * _D)],
        xbuf.at[pl.ds(_CH * _D, _CH * _D)], sem1)
    cp0.start()
    cp1.start()

    lanes = lax.iota(jnp.int32, _VEC)
    rows_off = lanes * _D

    def blk_body(b, carry):
        rows_b = rows_off + b * (_VEC * _D)
        accs = [jnp.zeros((_VEC,), jnp.float32) for _ in range(4)]
        for t in range(_D):
            c = (lanes + t) & (_D - 1)
            xv = plsc.load_gather(xbuf, [rows_b + c])
            accs[t % 4] = accs[t % 4] + plsc.load_gather(tbuf, [xv + c * _Q])
        res[pl.ds(b * _VEC, _VEC)] = (accs[0] + accs[1]) + (accs[2] + accs[3])
        return carry

    cp0.wait()
    lax.fori_loop(0, _CH // _VEC, blk_body, 0)
    cp1.wait()
    lax.fori_loop(_CH // _VEC, _RPW // _VEC, blk_body, 0)

    pltpu.sync_copy(res, out_hbm.at[pl.ds(base, _RPW)])


_sc_call = functools.partial(
    pl.kernel,
    out_type=jax.ShapeDtypeStruct((_B,), jnp.float32),
    mesh=plsc.VectorSubcoreMesh(core_axis_name="c", subcore_axis_name="s"),
    compiler_params=pltpu.CompilerParams(needs_layout_passes=False),
    scratch_types=[
        pltpu.VMEM((_RPW * _D,), jnp.int32),
        pltpu.VMEM((_D * _Q,), jnp.float32),
        pltpu.VMEM((_RPW,), jnp.float32),
        pltpu.SemaphoreType.DMA,
        pltpu.SemaphoreType.DMA,
    ],
)(_sc_body)


def kernel(x, category_parameters):
    lt = jnp.log(category_parameters * (1.0 - 0.2 * _Q) + 0.2).T
    out = _sc_call(x.astype(jnp.int32).reshape(-1),
                   lt.reshape(-1).astype(jnp.float32))
    return lax.stop_gradient(out[:, None])


# speedup vs baseline: 1.7845x; 1.0187x over previous
import functools

import jax
import jax.numpy as jnp
from jax import lax
from jax.experimental import pallas as pl
from jax.experimental.pallas import tpu as pltpu
from jax.experimental.pallas import tpu_sc as plsc

_Q = 4
_D = 128
_B = 16384
_NC = 2
_NS = 16
_NW = _NC * _NS
_RPW = _B // _NW
_VEC = 16
_CH = _RPW // 2


def _sc_body(x_hbm, lt_hbm, out_hbm, xbuf, tbuf, res, sem0, sem1):
    wid = lax.axis_index("s") * _NC + lax.axis_index("c")
    base = wid * _RPW
    pltpu.sync_copy(lt_hbm, tbuf)

    cp0 = pltpu.make_async_copy(
        x_hbm.at[pl.ds(base * _D, _CH * _D)],
        xbuf.at[pl.ds(0, _CH * _D)], sem0)
    cp1 = pltpu.make_async_copy(
        x_hbm.at[pl.ds((base + _CH) * _D, _CH * _D)],
        xbuf.at[pl.ds(_CH * _D, _CH * _D)], sem1)
    cp0.start()
    cp1.start()

    lanes = lax.iota(jnp.int32, _VEC)
    rows_off = lanes * _D

    def blk_body(b, carry):
        rows_b = rows_off + b * (_VEC * _D)
        accs = [jnp.zeros((_VEC,), jnp.float32) for _ in range(4)]
        cs = [lanes, (lanes + 1) & (_D - 1)]
        for t in range(_D):
            c = cs[t % 2]
            xv = plsc.load_gather(xbuf, [rows_b + c])
            accs[t % 4] = accs[t % 4] + plsc.load_gather(tbuf, [xv * _D + c])
            cs[t % 2] = (c + 2) & (_D - 1)
        res[pl.ds(b * _VEC, _VEC)] = (accs[0] + accs[1]) + (accs[2] + accs[3])
        return carry

    cp0.wait()
    lax.fori_loop(0, _CH // _VEC, blk_body, 0)
    cp1.wait()
    lax.fori_loop(_CH // _VEC, _RPW // _VEC, blk_body, 0)

    pltpu.sync_copy(res, out_hbm.at[pl.ds(base, _RPW)])


_sc_call = functools.partial(
    pl.kernel,
    out_type=jax.ShapeDtypeStruct((_B,), jnp.float32),
    mesh=plsc.VectorSubcoreMesh(core_axis_name="c", subcore_axis_name="s"),
    compiler_params=pltpu.CompilerParams(needs_layout_passes=False),
    scratch_types=[
        pltpu.VMEM((_RPW * _D,), jnp.int32),
        pltpu.VMEM((_D * _Q,), jnp.float32),
        pltpu.VMEM((_RPW,), jnp.float32),
        pltpu.SemaphoreType.DMA,
        pltpu.SemaphoreType.DMA,
    ],
)(_sc_body)


def kernel(x, category_parameters):
    lt = jnp.log(category_parameters * (1.0 - 0.2 * _Q) + 0.2)
    out = _sc_call(x.astype(jnp.int32).reshape(-1),
                   lt.reshape(-1).astype(jnp.float32))
    return lax.stop_gradient(out[:, None])


# speedup vs baseline: 1.9302x; 1.0816x over previous
import functools

import jax
import jax.numpy as jnp
from jax import lax
from jax.experimental import pallas as pl
from jax.experimental.pallas import tpu as pltpu
from jax.experimental.pallas import tpu_sc as plsc

_Q = 4
_D = 128
_B = 16384
_NC = 2
_NS = 16
_NW = _NC * _NS
_RPW = _B // _NW
_VEC = 16
_CH = _RPW // 2


def _sc_body(x_hbm, lt_hbm, out_hbm, xbuf, tbuf, res, sem0, sem1):
    wid = lax.axis_index("s") * _NC + lax.axis_index("c")
    base = wid * _RPW
    pltpu.sync_copy(lt_hbm, tbuf)

    cp0 = pltpu.make_async_copy(
        x_hbm.at[pl.ds(base * _D, _CH * _D)],
        xbuf.at[pl.ds(0, _CH * _D)], sem0)
    cp1 = pltpu.make_async_copy(
        x_hbm.at[pl.ds((base + _CH) * _D, _CH * _D)],
        xbuf.at[pl.ds(_CH * _D, _CH * _D)], sem1)
    cp0.start()
    cp1.start()

    lanes = lax.iota(jnp.int32, _VEC)
    rows_off = lanes * _D

    _UNROLL = 32

    def blk_body(b, carry):
        @pl.when(b == 0)
        def _():
            cp0.wait()

        @pl.when(b == _CH // _VEC)
        def _():
            cp1.wait()

        rows_b = rows_off + b * (_VEC * _D)

        def chunk_steps(u, accs):
            accs = list(accs)
            c0 = (lanes + u * _UNROLL) & (_D - 1)
            cs = [c0, (c0 + 1) & (_D - 1)]
            for t in range(_UNROLL):
                c = cs[t % 2]
                xv = plsc.load_gather(xbuf, [rows_b + c])
                accs[t % 4] = accs[t % 4] + plsc.load_gather(
                    tbuf, [xv * _D + c])
                cs[t % 2] = (c + 2) & (_D - 1)
            return tuple(accs)

        zero = jnp.zeros((_VEC,), jnp.float32)
        accs = lax.fori_loop(0, _D // _UNROLL, chunk_steps, (zero,) * 4)
        res[pl.ds(b * _VEC, _VEC)] = (accs[0] + accs[1]) + (accs[2] + accs[3])
        return carry

    lax.fori_loop(0, _RPW // _VEC, blk_body, 0)

    pltpu.sync_copy(res, out_hbm.at[pl.ds(base, _RPW)])


_sc_call = functools.partial(
    pl.kernel,
    out_type=jax.ShapeDtypeStruct((_B,), jnp.float32),
    mesh=plsc.VectorSubcoreMesh(core_axis_name="c", subcore_axis_name="s"),
    compiler_params=pltpu.CompilerParams(needs_layout_passes=False),
    scratch_types=[
        pltpu.VMEM((_RPW * _D,), jnp.int32),
        pltpu.VMEM((_D * _Q,), jnp.float32),
        pltpu.VMEM((_RPW,), jnp.float32),
        pltpu.SemaphoreType.DMA,
        pltpu.SemaphoreType.DMA,
    ],
)(_sc_body)


def kernel(x, category_parameters):
    lt = jnp.log(category_parameters * (1.0 - 0.2 * _Q) + 0.2)
    out = _sc_call(x.astype(jnp.int32).reshape(-1),
                   lt.reshape(-1).astype(jnp.float32))
    return lax.stop_gradient(out[:, None])


# speedup vs baseline: 2.2697x; 1.1759x over previous
import functools

import jax
import jax.numpy as jnp
from jax import lax
from jax.experimental import pallas as pl
from jax.experimental.pallas import tpu as pltpu
from jax.experimental.pallas import tpu_sc as plsc

_Q = 4
_D = 128
_B = 16384
_NC = 2
_NS = 16
_NW = _NC * _NS
_RPW = _B // _NW
_VEC = 16
_CH = _RPW // 2


def _sc_body(x_hbm, lt_hbm, out_hbm, xbuf, tbuf, res, sem0, sem1):
    wid = lax.axis_index("s") * _NC + lax.axis_index("c")
    base = wid * _RPW
    pltpu.sync_copy(lt_hbm, tbuf)

    cp0 = pltpu.make_async_copy(
        x_hbm.at[pl.ds(base * _D, _CH * _D)],
        xbuf.at[pl.ds(0, _CH * _D)], sem0)
    cp1 = pltpu.make_async_copy(
        x_hbm.at[pl.ds((base + _CH) * _D, _CH * _D)],
        xbuf.at[pl.ds(_CH * _D, _CH * _D)], sem1)
    cp0.start()
    cp1.start()

    lanes = lax.iota(jnp.int32, _VEC)
    rows_off = lanes * _D

    _UNROLL = 16

    def blk_body(b, carry):
        @pl.when(b == 0)
        def _():
            cp0.wait()

        @pl.when(b == _CH // _VEC)
        def _():
            cp1.wait()

        rows_b = rows_off + b * (_VEC * _D)

        def chunk_steps(u, accs):
            accs = list(accs)
            c0 = (lanes + u * _UNROLL) & (_D - 1)
            cs = [c0, (c0 + 1) & (_D - 1)]
            for t in range(_UNROLL):
                c = cs[t % 2]
                xv = plsc.load_gather(xbuf, [rows_b + c])
                accs[t % 4] = accs[t % 4] + plsc.load_gather(
                    tbuf, [xv * _D + c])
                cs[t % 2] = (c + 2) & (_D - 1)
            return tuple(accs)

        zero = jnp.zeros((_VEC,), jnp.float32)
        accs = lax.fori_loop(0, _D // _UNROLL, chunk_steps, (zero,) * 4)
        res[pl.ds(b * _VEC, _VEC)] = (accs[0] + accs[1]) + (accs[2] + accs[3])
        return carry

    lax.fori_loop(0, _RPW // _VEC, blk_body, 0)

    pltpu.sync_copy(res, out_hbm.at[pl.ds(base, _RPW)])


_sc_call = functools.partial(
    pl.kernel,
    out_type=jax.ShapeDtypeStruct((_B,), jnp.float32),
    mesh=plsc.VectorSubcoreMesh(core_axis_name="c", subcore_axis_name="s"),
    compiler_params=pltpu.CompilerParams(needs_layout_passes=False),
    scratch_types=[
        pltpu.VMEM((_RPW * _D,), jnp.int32),
        pltpu.VMEM((_D * _Q,), jnp.float32),
        pltpu.VMEM((_RPW,), jnp.float32),
        pltpu.SemaphoreType.DMA,
        pltpu.SemaphoreType.DMA,
    ],
)(_sc_body)


def kernel(x, category_parameters):
    lt = jnp.log(category_parameters * (1.0 - 0.2 * _Q) + 0.2)
    out = _sc_call(x.astype(jnp.int32).reshape(-1),
                   lt.reshape(-1).astype(jnp.float32))
    return lax.stop_gradient(out[:, None])
